# Initial kernel scaffold; baseline (speedup 1.0000x reference)
import functools

import jax
import jax.numpy as jnp
from jax import lax
from jax.experimental import pallas as pl
from jax.experimental.pallas import tpu as pltpu
from jax.experimental.pallas import tpu_sc as plsc

N = 10000
E = 320000
D = 128
C = 128

CA = C + 16
NC = 2
NS = 16
L = 16
NW = NC * NS
EPW = E // NW
K = 80
NCH = EPW // K
RPT = N // NS
KG = K // L


def _prep_body(x_ref, w_ref, s_ref, d_ref, xpa_ref, as_ref, ad_ref, ms_ref, md_ref):
    xp = jnp.dot(x_ref[...], w_ref[...], preferred_element_type=jnp.float32)
    xpa_ref[:, :C] = xp
    pat = jnp.where(lax.broadcasted_iota(jnp.int32, (N, CA - C), 1) == 0, 1.0, 0.0)
    xpa_ref[:, C:] = pat.astype(jnp.float32)
    a_s = jnp.sum(xp * s_ref[...][None, :], axis=1, keepdims=True)
    a_d = jnp.sum(xp * d_ref[...][None, :], axis=1, keepdims=True)
    as_ref[...] = a_s
    ad_ref[...] = a_d
    ms_ref[...] = jnp.full((1, 1), jnp.max(a_s), jnp.float32)
    md_ref[...] = jnp.full((1, 1), jnp.max(a_d), jnp.float32)


_prep_call = pl.pallas_call(
    _prep_body,
    out_shape=[
        jax.ShapeDtypeStruct((N, CA), jnp.float32),
        jax.ShapeDtypeStruct((N, 1), jnp.float32),
        jax.ShapeDtypeStruct((N, 1), jnp.float32),
        jax.ShapeDtypeStruct((1, 1), jnp.float32),
        jax.ShapeDtypeStruct((1, 1), jnp.float32),
    ],
)


def _sc_body(xpa_hbm, as_hbm, ad_hbm, m_hbm, src_hbm, dst_hbm, z_hbm,
             acc_hbm,
             as_v, ad_v, m_v, src_v, dst_v, ex_v, rows_v, acc_sh, sem):
    cid = lax.axis_index("c")
    sid = lax.axis_index("s")
    wid = cid * NS + sid

    pltpu.sync_copy(as_hbm, as_v)
    pltpu.sync_copy(ad_hbm, ad_v)
    pltpu.sync_copy(m_hbm, m_v)
    mvec = m_v[...]

    row0 = sid * RPT
    pltpu.sync_copy(z_hbm, acc_sh.at[pl.ds(row0, RPT)])
    plsc.subcore_barrier()

    ebase = wid * EPW

    def chunk_body(g, carry):
        base = ebase + g * K
        pltpu.sync_copy(src_hbm.at[pl.ds(base, K)], src_v)
        pltpu.sync_copy(dst_hbm.at[pl.ds(base, K)], dst_v)
        pltpu.async_copy(xpa_hbm.at[src_v], rows_v, sem).wait()
        for t in range(KG):
            sv = src_v[pl.ds(t * L, L)]
            dv = dst_v[pl.ds(t * L, L)]
            asv = plsc.load_gather(as_v, [sv])
            adv = plsc.load_gather(ad_v, [dv])
            e = asv + adv
            e = jnp.where(e >= 0.0, e, 0.2 * e)
            ex_v[...] = jnp.exp(e - mvec)
            for r in range(L):
                i = t * L + r
                bex = plsc.load_gather(ex_v, [jnp.full((L,), r, jnp.int32)])
                for j in range(CA // L):
                    rows_v[i, pl.ds(j * L, L)] = rows_v[i, pl.ds(j * L, L)] * bex
        pltpu.sync_copy(rows_v, acc_sh.at[dst_v], add=True)
        return carry

    lax.fori_loop(0, NCH, chunk_body, 0)

    plsc.subcore_barrier()
    pltpu.sync_copy(acc_sh.at[pl.ds(row0, RPT)], acc_hbm.at[cid, pl.ds(row0, RPT)])


_sc_call = pl.kernel(
    _sc_body,
    out_type=jax.ShapeDtypeStruct((NC, N, CA), jnp.float32),
    mesh=plsc.VectorSubcoreMesh(
        core_axis_name="c", subcore_axis_name="s", num_cores=NC, num_subcores=NS
    ),
    scratch_types=[
        pltpu.VMEM((N,), jnp.float32),
        pltpu.VMEM((N,), jnp.float32),
        pltpu.VMEM((L,), jnp.float32),
        pltpu.VMEM((K,), jnp.int32),
        pltpu.VMEM((K,), jnp.int32),
        pltpu.VMEM((L,), jnp.float32),
        pltpu.VMEM((K, CA), jnp.float32),
        pltpu.VMEM_SHARED((N, CA), jnp.float32),
        pltpu.SemaphoreType.DMA,
    ],
)


def _final_body(acc_ref, bias_ref, out_ref):
    a = acc_ref[0] + acc_ref[1]
    den = a[:, C:C + 1]
    o = a[:, :C] / (den + 1e-16) + bias_ref[...][None, :]
    m = jnp.max(o, axis=1, keepdims=True)
    ls = o - m
    out_ref[...] = ls - jnp.log(jnp.sum(jnp.exp(ls), axis=1, keepdims=True))


_final_call = pl.pallas_call(
    _final_body,
    out_shape=jax.ShapeDtypeStruct((N, C), jnp.float32),
)


def kernel(x, W, att_src, att_dst, bias, edge_index):
    xpa, as2, ad2, ms, md = _prep_call(x, W, att_src, att_dst)
    a_s = as2.reshape(N)
    a_d = ad2.reshape(N)
    mm = ms[0, 0] + md[0, 0]
    M = jnp.where(mm >= 0.0, mm, 0.2 * mm)
    m16 = jnp.full((L,), M, jnp.float32)
    src = edge_index[0]
    dst = edge_index[1]
    zrows = jnp.zeros((RPT, CA), jnp.float32)
    acc = _sc_call(xpa, a_s, a_d, m16, src, dst, zrows)
    return _final_call(acc, bias)


# speedup vs baseline: 9.2502x; 9.2502x over previous
import jax
import jax.numpy as jnp
from jax import lax
from jax.experimental import pallas as pl
from jax.experimental.pallas import tpu as pltpu
from jax.experimental.pallas import tpu_sc as plsc

N = 10000
E = 320000
D = 128
C = 128

NC = 2
NS = 16
L = 16
NW = NC * NS
EPW = E // NW
K = 16
NCH = EPW // K
NB = 5
NOUT = NCH // NB
RPT = 624
REM = N - NS * RPT
TAIL = NS * RPT
KG = K // L
PR = 208
PH = NS * PR
NPH = 3
RV = PR // L


def _prep_body(x_ref, w_ref, s_ref, d_ref, xp_ref, as_ref, ad_ref, ms_ref, md_ref):
    xp = jnp.dot(x_ref[...], w_ref[...], preferred_element_type=jnp.float32)
    xp_ref[...] = xp
    a_s = jnp.sum(xp * s_ref[...][None, :], axis=1, keepdims=True)
    a_d = jnp.sum(xp * d_ref[...][None, :], axis=1, keepdims=True)
    as_ref[...] = a_s
    ad_ref[...] = a_d
    ms_ref[...] = jnp.full((1, 1), jnp.max(a_s), jnp.float32)
    md_ref[...] = jnp.full((1, 1), jnp.max(a_d), jnp.float32)


_prep_call = pl.pallas_call(
    _prep_body,
    out_shape=[
        jax.ShapeDtypeStruct((N, C), jnp.float32),
        jax.ShapeDtypeStruct((N, 1), jnp.float32),
        jax.ShapeDtypeStruct((N, 1), jnp.float32),
        jax.ShapeDtypeStruct((1, 1), jnp.float32),
        jax.ShapeDtypeStruct((1, 1), jnp.float32),
    ],
)


def _sc_body(xp_hbm, as_hbm, ad_hbm, m_hbm, src_hbm, dst_hbm, z_hbm, zn_hbm,
             acc_hbm, den_hbm,
             as_v, ad_v, m_v, src_v, dst_v, ex_v, rows_v, denom_v,
             dacc_v, dstage_v, t0_v, t1_v, acc_sh, den_sh, sems):
    cid = lax.axis_index("c")
    sid = lax.axis_index("s")
    wid = cid * NS + sid

    pltpu.sync_copy(as_hbm, as_v)
    pltpu.sync_copy(ad_hbm, ad_v)
    pltpu.sync_copy(m_hbm, m_v)
    pltpu.sync_copy(zn_hbm, denom_v)
    mvec = m_v[...]

    row0 = sid * RPT
    pltpu.sync_copy(z_hbm.at[pl.ds(0, RPT)], acc_sh.at[pl.ds(row0, RPT)])

    @pl.when(sid == 0)
    def _zero_tail():
        pltpu.sync_copy(z_hbm.at[pl.ds(0, REM)], acc_sh.at[pl.ds(TAIL, REM)])

    plsc.subcore_barrier()

    ebase = wid * EPW

    def start_gather(g, b):
        base = ebase + lax.rem(g, NCH) * K
        pltpu.sync_copy(src_hbm.at[pl.ds(base, K)], src_v.at[b])
        pltpu.sync_copy(dst_hbm.at[pl.ds(base, K)], dst_v.at[b])
        return pltpu.async_copy(xp_hbm.at[src_v.at[b]], rows_v.at[b], sems.at[b])

    def scale(b):
        for t in range(KG):
            sv = src_v[b, pl.ds(t * L, L)]
            dv = dst_v[b, pl.ds(t * L, L)]
            asv = plsc.load_gather(as_v, [sv])
            adv = plsc.load_gather(ad_v, [dv])
            e = asv + adv
            e = jnp.where(e >= 0.0, e, 0.2 * e)
            ex = jnp.exp(e - mvec)
            ex_v[pl.ds(L, L)] = ex
            plsc.addupdate_scatter(denom_v, [dv], ex)
            for r in range(L):
                i = t * L + r
                bex = plsc.load_gather(ex_v, [jnp.full((L,), L + r, jnp.int32)])
                for j in range(C // L):
                    rows_v[b, i, pl.ds(j * L, L)] = (
                        rows_v[b, i, pl.ds(j * L, L)] * bex)

    start_gather(0, 0).wait()
    scale(0)

    def outer(G, carry):
        for b in range(NB):
            g = G * NB + b
            start_gather(g + 1, (b + 1) % NB).wait()
            scale((b + 1) % NB)
            pltpu.sync_copy(rows_v.at[b], acc_sh.at[dst_v.at[b]], add=True)
        return carry

    lax.fori_loop(0, NOUT - 1, outer, 0)
    base_last = (NOUT - 1) * NB
    for b in range(NB - 1):
        start_gather(base_last + b + 1, (b + 1) % NB).wait()
        scale((b + 1) % NB)
        pltpu.sync_copy(rows_v.at[b], acc_sh.at[dst_v.at[b]], add=True)
    pltpu.sync_copy(rows_v.at[NB - 1], acc_sh.at[dst_v.at[NB - 1]], add=True)

    plsc.subcore_barrier()

    pltpu.sync_copy(acc_sh.at[pl.ds(row0, RPT)], acc_hbm.at[cid, pl.ds(row0, RPT)])

    @pl.when(sid == 0)
    def _write_tail():
        pltpu.sync_copy(acc_sh.at[pl.ds(TAIL, REM)],
                        acc_hbm.at[cid, pl.ds(TAIL, REM)])

    for p in range(NPH):
        pb = p * PH
        pltpu.sync_copy(denom_v.at[pl.ds(pb, PH)], den_sh.at[pl.ds(sid * PH, PH)])
        plsc.subcore_barrier()
        my0 = pb + sid * PR
        pltpu.sync_copy(den_sh.at[pl.ds(sid * PR, PR)], dacc_v)
        for k in range(1, NS):
            pltpu.sync_copy(den_sh.at[pl.ds(k * PH + sid * PR, PR)], dstage_v)

            def add_body(i, carry):
                dacc_v[pl.ds(i * L, L)] = (
                    dacc_v[pl.ds(i * L, L)] + dstage_v[pl.ds(i * L, L)])
                return carry

            lax.fori_loop(0, RV, add_body, 0)
        pltpu.sync_copy(dacc_v, den_hbm.at[pl.ds(cid * N + my0, PR)])
        plsc.subcore_barrier()

    pltpu.sync_copy(denom_v.at[pl.ds(TAIL, REM)], den_sh.at[pl.ds(sid * REM, REM)])
    plsc.subcore_barrier()

    @pl.when(sid == 0)
    def _den_tail():
        pltpu.sync_copy(den_sh.at[pl.ds(0, REM)], t0_v)
        for k in range(1, NS):
            pltpu.sync_copy(den_sh.at[pl.ds(k * REM, REM)], t1_v)
            t0_v[...] = t0_v[...] + t1_v[...]
        pltpu.sync_copy(t0_v, den_hbm.at[pl.ds(cid * N + TAIL, REM)])


_sc_call = pl.kernel(
    _sc_body,
    out_type=[
        jax.ShapeDtypeStruct((NC, N, C), jnp.float32),
        jax.ShapeDtypeStruct((NC * N,), jnp.float32),
    ],
    mesh=plsc.VectorSubcoreMesh(
        core_axis_name="c", subcore_axis_name="s", num_cores=NC, num_subcores=NS
    ),
    compiler_params=pltpu.CompilerParams(needs_layout_passes=False),
    scratch_types=[
        pltpu.VMEM((N,), jnp.float32),
        pltpu.VMEM((N,), jnp.float32),
        pltpu.VMEM((L,), jnp.float32),
        pltpu.VMEM((NB, K), jnp.int32),
        pltpu.VMEM((NB, K), jnp.int32),
        pltpu.VMEM((2 * L,), jnp.float32),
        pltpu.VMEM((NB, K, C), jnp.float32),
        pltpu.VMEM((N,), jnp.float32),
        pltpu.VMEM((PR,), jnp.float32),
        pltpu.VMEM((PR,), jnp.float32),
        pltpu.VMEM((L,), jnp.float32),
        pltpu.VMEM((L,), jnp.float32),
        pltpu.VMEM_SHARED((N, C), jnp.float32),
        pltpu.VMEM_SHARED((NS * PH,), jnp.float32),
        pltpu.SemaphoreType.DMA((NB,)),
    ],
)


def _final_body(acc_ref, den_ref, bias_ref, out_ref):
    a = acc_ref[0] + acc_ref[1]
    den = den_ref[0] + den_ref[1]
    o = a / (den + 1e-16) + bias_ref[...][None, :]
    m = jnp.max(o, axis=1, keepdims=True)
    ls = o - m
    out_ref[...] = ls - jnp.log(jnp.sum(jnp.exp(ls), axis=1, keepdims=True))


_final_call = pl.pallas_call(
    _final_body,
    out_shape=jax.ShapeDtypeStruct((N, C), jnp.float32),
)


def kernel(x, W, att_src, att_dst, bias, edge_index):
    xp, as2, ad2, ms, md = _prep_call(x, W, att_src, att_dst)
    a_s = as2.reshape(N)
    a_d = ad2.reshape(N)
    mm = ms[0, 0] + md[0, 0]
    M = jnp.where(mm >= 0.0, mm, 0.2 * mm)
    m16 = jnp.full((L,), M, jnp.float32)
    src = edge_index[0]
    dst = edge_index[1]
    zrows = jnp.zeros((RPT, C), jnp.float32)
    zn = jnp.zeros((N,), jnp.float32)
    acc, den = _sc_call(xp, a_s, a_d, m16, src, dst, zrows, zn)
    den3 = den.reshape(NC, N, 1)
    return _final_call(acc, den3, bias)


# speedup vs baseline: 28.1084x; 3.0387x over previous
import jax
import jax.numpy as jnp
from jax import lax
from jax.experimental import pallas as pl
from jax.experimental.pallas import tpu as pltpu
from jax.experimental.pallas import tpu_sc as plsc

N = 10000
E = 320000
D = 128
C = 128

NC = 2
NS = 16
L = 16
NW = NC * NS
EPW = E // NW
K = 16
NCH = EPW // K
NB = 5
SB = 125
NSB = NCH // SB
RPT = 624
REM = N - NS * RPT
TAIL = NS * RPT
KG = K // L
PR = 208
PH = NS * PR
NPH = 3
RV = PR // L


def _prep_body(x_ref, w_ref, s_ref, d_ref, xp_ref, as_ref, ad_ref, ms_ref, md_ref):
    xp = jnp.dot(x_ref[...], w_ref[...], preferred_element_type=jnp.float32)
    xp_ref[...] = xp
    a_s = jnp.sum(xp * s_ref[...][None, :], axis=1, keepdims=True)
    a_d = jnp.sum(xp * d_ref[...][None, :], axis=1, keepdims=True)
    as_ref[...] = a_s
    ad_ref[...] = a_d
    ms_ref[...] = jnp.full((1, 1), jnp.max(a_s), jnp.float32)
    md_ref[...] = jnp.full((1, 1), jnp.max(a_d), jnp.float32)


_prep_call = pl.pallas_call(
    _prep_body,
    out_shape=[
        jax.ShapeDtypeStruct((N, C), jnp.float32),
        jax.ShapeDtypeStruct((N, 1), jnp.float32),
        jax.ShapeDtypeStruct((N, 1), jnp.float32),
        jax.ShapeDtypeStruct((1, 1), jnp.float32),
        jax.ShapeDtypeStruct((1, 1), jnp.float32),
    ],
)


def _sc_body(xp_hbm, as_hbm, ad_hbm, m_hbm, src_hbm, dst_hbm, z_hbm, zn_hbm,
             acc_hbm, den_hbm,
             as_v, ad_v, m_v, src_blk, dst_blk, dst_v, ex_v, rows_v, denom_v,
             dacc_v, dstage_v, t0_v, t1_v, acc_sh, den_sh, sems):
    cid = lax.axis_index("c")
    sid = lax.axis_index("s")
    wid = cid * NS + sid

    pltpu.sync_copy(as_hbm, as_v)
    pltpu.sync_copy(ad_hbm, ad_v)
    pltpu.sync_copy(m_hbm, m_v)
    pltpu.sync_copy(zn_hbm, denom_v)
    mvec = m_v[...]

    row0 = sid * RPT
    pltpu.sync_copy(z_hbm.at[pl.ds(0, RPT)], acc_sh.at[pl.ds(row0, RPT)])

    @pl.when(sid == 0)
    def _zero_tail():
        pltpu.sync_copy(z_hbm.at[pl.ds(0, REM)], acc_sh.at[pl.ds(TAIL, REM)])

    plsc.subcore_barrier()

    ebase = wid * EPW

    def start_gather(c, b):
        return pltpu.async_copy(
            xp_hbm.at[src_blk.at[pl.ds(c * K, K)]], rows_v.at[b], sems.at[b])

    def wait_gather(c, b):
        pltpu.make_async_copy(
            xp_hbm.at[src_blk.at[pl.ds(c * K, K)]], rows_v.at[b],
            sems.at[b]).wait()

    def scale(c, b):
        sv = src_blk[pl.ds(c * K, L)]
        dv = dst_blk[pl.ds(c * K, L)]
        dst_v[b, :] = dv
        asv = plsc.load_gather(as_v, [sv])
        adv = plsc.load_gather(ad_v, [dv])
        e = asv + adv
        e = jnp.where(e >= 0.0, e, 0.2 * e)
        ex = jnp.exp(e - mvec)
        ex_v[pl.ds(L, L)] = ex
        plsc.addupdate_scatter(denom_v, [dv], ex)
        for r in range(L):
            bex = plsc.load_gather(ex_v, [jnp.full((L,), L + r, jnp.int32)])
            for j in range(C // L):
                rows_v[b, r, pl.ds(j * L, L)] = (
                    rows_v[b, r, pl.ds(j * L, L)] * bex)

    def scatter(b):
        pltpu.sync_copy(rows_v.at[b], acc_sh.at[dst_v.at[b]], add=True)

    def superblock(S, carry):
        sbase = ebase + S * SB * K
        pltpu.sync_copy(src_hbm.at[pl.ds(sbase, SB * K)], src_blk)
        pltpu.sync_copy(dst_hbm.at[pl.ds(sbase, SB * K)], dst_blk)
        start_gather(0, 0)
        start_gather(1, 1)

        def inner(I, carry2):
            for jj in range(NB):
                c = I * NB + jj
                wait_gather(c, jj)
                scale(c, jj)

                @pl.when(c + 2 < SB)
                def _pf():
                    start_gather(c + 2, (jj + 2) % NB)

                if jj == 0:
                    @pl.when(I > 0)
                    def _sc0():
                        scatter(NB - 1)
                else:
                    scatter(jj - 1)
            return carry2

        lax.fori_loop(0, SB // NB, inner, 0)
        scatter(NB - 1)
        return carry

    lax.fori_loop(0, NSB, superblock, 0)

    plsc.subcore_barrier()

    pltpu.sync_copy(acc_sh.at[pl.ds(row0, RPT)], acc_hbm.at[cid, pl.ds(row0, RPT)])

    @pl.when(sid == 0)
    def _write_tail():
        pltpu.sync_copy(acc_sh.at[pl.ds(TAIL, REM)],
                        acc_hbm.at[cid, pl.ds(TAIL, REM)])

    for p in range(NPH):
        pb = p * PH
        pltpu.sync_copy(denom_v.at[pl.ds(pb, PH)], den_sh.at[pl.ds(sid * PH, PH)])
        plsc.subcore_barrier()
        my0 = pb + sid * PR
        pltpu.sync_copy(den_sh.at[pl.ds(sid * PR, PR)], dacc_v)
        for k in range(1, NS):
            pltpu.sync_copy(den_sh.at[pl.ds(k * PH + sid * PR, PR)], dstage_v)

            def add_body(i, carry):
                dacc_v[pl.ds(i * L, L)] = (
                    dacc_v[pl.ds(i * L, L)] + dstage_v[pl.ds(i * L, L)])
                return carry

            lax.fori_loop(0, RV, add_body, 0)
        pltpu.sync_copy(dacc_v, den_hbm.at[pl.ds(cid * N + my0, PR)])
        plsc.subcore_barrier()

    pltpu.sync_copy(denom_v.at[pl.ds(TAIL, REM)], den_sh.at[pl.ds(sid * REM, REM)])
    plsc.subcore_barrier()

    @pl.when(sid == 0)
    def _den_tail():
        pltpu.sync_copy(den_sh.at[pl.ds(0, REM)], t0_v)
        for k in range(1, NS):
            pltpu.sync_copy(den_sh.at[pl.ds(k * REM, REM)], t1_v)
            t0_v[...] = t0_v[...] + t1_v[...]
        pltpu.sync_copy(t0_v, den_hbm.at[pl.ds(cid * N + TAIL, REM)])


_sc_call = pl.kernel(
    _sc_body,
    out_type=[
        jax.ShapeDtypeStruct((NC, N, C), jnp.float32),
        jax.ShapeDtypeStruct((NC * N,), jnp.float32),
    ],
    mesh=plsc.VectorSubcoreMesh(
        core_axis_name="c", subcore_axis_name="s", num_cores=NC, num_subcores=NS
    ),
    compiler_params=pltpu.CompilerParams(needs_layout_passes=False),
    scratch_types=[
        pltpu.VMEM((N,), jnp.float32),
        pltpu.VMEM((N,), jnp.float32),
        pltpu.VMEM((L,), jnp.float32),
        pltpu.VMEM((SB * K,), jnp.int32),
        pltpu.VMEM((SB * K,), jnp.int32),
        pltpu.VMEM((NB, K), jnp.int32),
        pltpu.VMEM((2 * L,), jnp.float32),
        pltpu.VMEM((NB, K, C), jnp.float32),
        pltpu.VMEM((N,), jnp.float32),
        pltpu.VMEM((PR,), jnp.float32),
        pltpu.VMEM((PR,), jnp.float32),
        pltpu.VMEM((L,), jnp.float32),
        pltpu.VMEM((L,), jnp.float32),
        pltpu.VMEM_SHARED((N, C), jnp.float32),
        pltpu.VMEM_SHARED((NS * PH,), jnp.float32),
        pltpu.SemaphoreType.DMA((NB,)),
    ],
)


def _final_body(acc_ref, den_ref, bias_ref, out_ref):
    a = acc_ref[0] + acc_ref[1]
    den = den_ref[0] + den_ref[1]
    o = a / (den + 1e-16) + bias_ref[...][None, :]
    m = jnp.max(o, axis=1, keepdims=True)
    ls = o - m
    out_ref[...] = ls - jnp.log(jnp.sum(jnp.exp(ls), axis=1, keepdims=True))


_final_call = pl.pallas_call(
    _final_body,
    out_shape=jax.ShapeDtypeStruct((N, C), jnp.float32),
)


def kernel(x, W, att_src, att_dst, bias, edge_index):
    xp, as2, ad2, ms, md = _prep_call(x, W, att_src, att_dst)
    a_s = as2.reshape(N)
    a_d = ad2.reshape(N)
    mm = ms[0, 0] + md[0, 0]
    M = jnp.where(mm >= 0.0, mm, 0.2 * mm)
    m16 = jnp.full((L,), M, jnp.float32)
    src = edge_index[0]
    dst = edge_index[1]
    zrows = jnp.zeros((RPT, C), jnp.float32)
    zn = jnp.zeros((N,), jnp.float32)
    acc, den = _sc_call(xp, a_s, a_d, m16, src, dst, zrows, zn)
    den3 = den.reshape(NC, N, 1)
    return _final_call(acc, den3, bias)


# speedup vs baseline: 28.8661x; 1.0270x over previous
import jax
import jax.numpy as jnp
from jax import lax
from jax.experimental import pallas as pl
from jax.experimental.pallas import tpu as pltpu
from jax.experimental.pallas import tpu_sc as plsc

N = 10000
E = 320000
D = 128
C = 128

NC = 2
NS = 16
L = 16
NW = NC * NS
EPW = E // NW
K = 16
NCH = EPW // K
NB = 5
SB = 125
NSB = NCH // SB
RPT = 624
REM = N - NS * RPT
TAIL = NS * RPT
KG = K // L
PR = 208
PH = NS * PR
NPH = 3
RV = PR // L


def _prep_body(x_ref, w_ref, s_ref, d_ref, xp_ref, as_ref, ad_ref, ms_ref, md_ref):
    xp = jnp.dot(x_ref[...], w_ref[...], preferred_element_type=jnp.float32)
    xp_ref[...] = xp
    a_s = jnp.sum(xp * s_ref[...][None, :], axis=1, keepdims=True)
    a_d = jnp.sum(xp * d_ref[...][None, :], axis=1, keepdims=True)
    as_ref[...] = a_s
    ad_ref[...] = a_d
    ms_ref[...] = jnp.full((1, 1), jnp.max(a_s), jnp.float32)
    md_ref[...] = jnp.full((1, 1), jnp.max(a_d), jnp.float32)


_prep_call = pl.pallas_call(
    _prep_body,
    out_shape=[
        jax.ShapeDtypeStruct((N, C), jnp.float32),
        jax.ShapeDtypeStruct((N, 1), jnp.float32),
        jax.ShapeDtypeStruct((N, 1), jnp.float32),
        jax.ShapeDtypeStruct((1, 1), jnp.float32),
        jax.ShapeDtypeStruct((1, 1), jnp.float32),
    ],
)


def _sc_body(xp_hbm, as_hbm, ad_hbm, m_hbm, src_hbm, dst_hbm, z_hbm, zn_hbm,
             acc_hbm, den_hbm,
             as_v, ad_v, m_v, src_blk, dst_blk, dst_v, ex_v, rows_v, denom_v,
             dacc_v, dstage_v, t0_v, t1_v, acc_sh, den_sh, sems, ssem):
    cid = lax.axis_index("c")
    sid = lax.axis_index("s")
    wid = cid * NS + sid

    pltpu.sync_copy(as_hbm, as_v)
    pltpu.sync_copy(ad_hbm, ad_v)
    pltpu.sync_copy(m_hbm, m_v)
    pltpu.sync_copy(zn_hbm, denom_v)
    mvec = m_v[...]

    row0 = sid * RPT
    pltpu.sync_copy(z_hbm.at[pl.ds(0, RPT)], acc_sh.at[pl.ds(row0, RPT)])

    @pl.when(sid == 0)
    def _zero_tail():
        pltpu.sync_copy(z_hbm.at[pl.ds(0, REM)], acc_sh.at[pl.ds(TAIL, REM)])

    plsc.subcore_barrier()

    ebase = wid * EPW

    def start_gather(c, b):
        return pltpu.async_copy(
            xp_hbm.at[src_blk.at[pl.ds(c * K, K)]], rows_v.at[b], sems.at[b])

    def wait_gather(c, b):
        pltpu.make_async_copy(
            xp_hbm.at[src_blk.at[pl.ds(c * K, K)]], rows_v.at[b],
            sems.at[b]).wait()

    def scale(c, b):
        sv = src_blk[pl.ds(c * K, L)]
        dv = dst_blk[pl.ds(c * K, L)]
        dst_v[b, :] = dv
        asv = plsc.load_gather(as_v, [sv])
        adv = plsc.load_gather(ad_v, [dv])
        e = asv + adv
        e = jnp.where(e >= 0.0, e, 0.2 * e)
        ex = jnp.exp(e - mvec)
        ex_v[pl.ds(L, L)] = ex
        plsc.addupdate_scatter(denom_v, [dv], ex)
        for r in range(L):
            bex = plsc.load_gather(ex_v, [jnp.full((L,), L + r, jnp.int32)])
            for j in range(C // L):
                rows_v[b, r, pl.ds(j * L, L)] = (
                    rows_v[b, r, pl.ds(j * L, L)] * bex)

    def start_scatter(b):
        pltpu.async_copy(rows_v.at[b], acc_sh.at[dst_v.at[b]], ssem, add=True)

    def wait_scatter():
        pltpu.make_async_copy(rows_v.at[0], acc_sh.at[dst_v.at[0]], ssem).wait()

    def superblock(S, carry):
        sbase = ebase + S * SB * K
        pltpu.sync_copy(src_hbm.at[pl.ds(sbase, SB * K)], src_blk)
        pltpu.sync_copy(dst_hbm.at[pl.ds(sbase, SB * K)], dst_blk)
        start_gather(0, 0)
        start_gather(1, 1)

        def inner(I, carry2):
            for jj in range(NB):
                c = I * NB + jj
                wait_gather(c, jj)
                scale(c, jj)

                @pl.when(c + 2 < SB)
                def _pf():
                    start_gather(c + 2, (jj + 2) % NB)

                if jj == 0:
                    @pl.when(I > 0)
                    def _sc0():
                        wait_scatter()
                        start_scatter(NB - 1)
                elif jj == 1:
                    @pl.when((S > 0) | (I > 0))
                    def _w1():
                        wait_scatter()
                    start_scatter(0)
                else:
                    wait_scatter()
                    start_scatter(jj - 1)
            return carry2

        lax.fori_loop(0, SB // NB, inner, 0)
        wait_scatter()
        start_scatter(NB - 1)
        return carry

    lax.fori_loop(0, NSB, superblock, 0)
    wait_scatter()

    plsc.subcore_barrier()

    pltpu.sync_copy(acc_sh.at[pl.ds(row0, RPT)], acc_hbm.at[cid, pl.ds(row0, RPT)])

    @pl.when(sid == 0)
    def _write_tail():
        pltpu.sync_copy(acc_sh.at[pl.ds(TAIL, REM)],
                        acc_hbm.at[cid, pl.ds(TAIL, REM)])

    for p in range(NPH):
        pb = p * PH
        pltpu.sync_copy(denom_v.at[pl.ds(pb, PH)], den_sh.at[pl.ds(sid * PH, PH)])
        plsc.subcore_barrier()
        my0 = pb + sid * PR
        pltpu.sync_copy(den_sh.at[pl.ds(sid * PR, PR)], dacc_v)
        for k in range(1, NS):
            pltpu.sync_copy(den_sh.at[pl.ds(k * PH + sid * PR, PR)], dstage_v)

            def add_body(i, carry):
                dacc_v[pl.ds(i * L, L)] = (
                    dacc_v[pl.ds(i * L, L)] + dstage_v[pl.ds(i * L, L)])
                return carry

            lax.fori_loop(0, RV, add_body, 0)
        pltpu.sync_copy(dacc_v, den_hbm.at[pl.ds(cid * N + my0, PR)])
        plsc.subcore_barrier()

    pltpu.sync_copy(denom_v.at[pl.ds(TAIL, REM)], den_sh.at[pl.ds(sid * REM, REM)])
    plsc.subcore_barrier()

    @pl.when(sid == 0)
    def _den_tail():
        pltpu.sync_copy(den_sh.at[pl.ds(0, REM)], t0_v)
        for k in range(1, NS):
            pltpu.sync_copy(den_sh.at[pl.ds(k * REM, REM)], t1_v)
            t0_v[...] = t0_v[...] + t1_v[...]
        pltpu.sync_copy(t0_v, den_hbm.at[pl.ds(cid * N + TAIL, REM)])


_sc_call = pl.kernel(
    _sc_body,
    out_type=[
        jax.ShapeDtypeStruct((NC, N, C), jnp.float32),
        jax.ShapeDtypeStruct((NC * N,), jnp.float32),
    ],
    mesh=plsc.VectorSubcoreMesh(
        core_axis_name="c", subcore_axis_name="s", num_cores=NC, num_subcores=NS
    ),
    compiler_params=pltpu.CompilerParams(needs_layout_passes=False),
    scratch_types=[
        pltpu.VMEM((N,), jnp.float32),
        pltpu.VMEM((N,), jnp.float32),
        pltpu.VMEM((L,), jnp.float32),
        pltpu.VMEM((SB * K,), jnp.int32),
        pltpu.VMEM((SB * K,), jnp.int32),
        pltpu.VMEM((NB, K), jnp.int32),
        pltpu.VMEM((2 * L,), jnp.float32),
        pltpu.VMEM((NB, K, C), jnp.float32),
        pltpu.VMEM((N,), jnp.float32),
        pltpu.VMEM((PR,), jnp.float32),
        pltpu.VMEM((PR,), jnp.float32),
        pltpu.VMEM((L,), jnp.float32),
        pltpu.VMEM((L,), jnp.float32),
        pltpu.VMEM_SHARED((N, C), jnp.float32),
        pltpu.VMEM_SHARED((NS * PH,), jnp.float32),
        pltpu.SemaphoreType.DMA((NB,)),
        pltpu.SemaphoreType.DMA,
    ],
)


def _final_body(acc_ref, den_ref, bias_ref, out_ref):
    a = acc_ref[0] + acc_ref[1]
    den = den_ref[0] + den_ref[1]
    o = a / (den + 1e-16) + bias_ref[...][None, :]
    m = jnp.max(o, axis=1, keepdims=True)
    ls = o - m
    out_ref[...] = ls - jnp.log(jnp.sum(jnp.exp(ls), axis=1, keepdims=True))


_final_call = pl.pallas_call(
    _final_body,
    out_shape=jax.ShapeDtypeStruct((N, C), jnp.float32),
)


def kernel(x, W, att_src, att_dst, bias, edge_index):
    xp, as2, ad2, ms, md = _prep_call(x, W, att_src, att_dst)
    a_s = as2.reshape(N)
    a_d = ad2.reshape(N)
    mm = ms[0, 0] + md[0, 0]
    M = jnp.where(mm >= 0.0, mm, 0.2 * mm)
    m16 = jnp.full((L,), M, jnp.float32)
    src = edge_index[0]
    dst = edge_index[1]
    zrows = jnp.zeros((RPT, C), jnp.float32)
    zn = jnp.zeros((N,), jnp.float32)
    acc, den = _sc_call(xp, a_s, a_d, m16, src, dst, zrows, zn)
    den3 = den.reshape(NC, N, 1)
    return _final_call(acc, den3, bias)


# speedup vs baseline: 28.9476x; 1.0028x over previous
import jax
import jax.numpy as jnp
from jax import lax
from jax.experimental import pallas as pl
from jax.experimental.pallas import tpu as pltpu
from jax.experimental.pallas import tpu_sc as plsc

N = 10000
E = 320000
D = 128
C = 128

NC = 2
NS = 16
L = 16
NW = NC * NS
EPW = E // NW
K = 16
NCH = EPW // K
NB = 5
SB = 125
NSB = NCH // SB
RPT = 624
REM = N - NS * RPT
TAIL = NS * RPT
PR = 208
PH = NS * PR
NPH = 3
RV = PR // L


def _prep_body(x_ref, w_ref, s_ref, d_ref, xp_ref, as_ref, ad_ref, ms_ref, md_ref):
    xp = jnp.dot(x_ref[...], w_ref[...], preferred_element_type=jnp.float32)
    xp_ref[...] = xp
    a_s = jnp.sum(xp * s_ref[...][None, :], axis=1, keepdims=True)
    a_d = jnp.sum(xp * d_ref[...][None, :], axis=1, keepdims=True)
    as_ref[...] = a_s
    ad_ref[...] = a_d
    ms_ref[...] = jnp.full((1, 1), jnp.max(a_s), jnp.float32)
    md_ref[...] = jnp.full((1, 1), jnp.max(a_d), jnp.float32)


_prep_call = pl.pallas_call(
    _prep_body,
    out_shape=[
        jax.ShapeDtypeStruct((N, C), jnp.float32),
        jax.ShapeDtypeStruct((N, 1), jnp.float32),
        jax.ShapeDtypeStruct((N, 1), jnp.float32),
        jax.ShapeDtypeStruct((1, 1), jnp.float32),
        jax.ShapeDtypeStruct((1, 1), jnp.float32),
    ],
)


def _sc_body(xp_hbm, as_hbm, ad_hbm, m_hbm, src_hbm, dst_hbm, z_hbm, zn_hbm,
             acc_hbm, den_hbm,
             as_v, ad_v, m_v, src_blk, dst_blk, dst_v, ex_v, rows_v, denom_v,
             dacc_v, dstage_v, t0_v, t1_v, acc_sh, den_sh, sems, ssem):
    cid = lax.axis_index("c")
    sid = lax.axis_index("s")
    wid = cid * NS + sid

    pltpu.sync_copy(as_hbm, as_v)
    pltpu.sync_copy(ad_hbm, ad_v)
    pltpu.sync_copy(m_hbm, m_v)
    pltpu.sync_copy(zn_hbm, denom_v)
    mvec = m_v[...]

    row0 = sid * RPT
    pltpu.sync_copy(z_hbm.at[pl.ds(0, RPT)], acc_sh.at[pl.ds(row0, RPT)])

    @pl.when(sid == 0)
    def _zero_tail():
        pltpu.sync_copy(z_hbm.at[pl.ds(0, REM)], acc_sh.at[pl.ds(TAIL, REM)])

    plsc.subcore_barrier()

    ebase = wid * EPW

    def start_gather(c, b):
        return pltpu.async_copy(
            xp_hbm.at[src_blk.at[pl.ds(c * K, K)]], rows_v.at[b], sems.at[b])

    def wait_gather(c, b):
        pltpu.make_async_copy(
            xp_hbm.at[src_blk.at[pl.ds(c * K, K)]], rows_v.at[b],
            sems.at[b]).wait()

    def scale(c, b):
        sv = src_blk[pl.ds(c * K, L)]
        dv = dst_blk[pl.ds(c * K, L)]
        dst_v[b, :] = dv
        asv = plsc.load_gather(as_v, [sv])
        adv = plsc.load_gather(ad_v, [dv])
        e = asv + adv
        e = jnp.where(e >= 0.0, e, 0.2 * e)
        ex = jnp.exp(e - mvec)
        ex_v[pl.ds(L, L)] = ex
        plsc.addupdate_scatter(denom_v, [dv], ex)
        for r in range(L):
            bex = plsc.load_gather(ex_v, [jnp.full((L,), L + r, jnp.int32)])
            for j in range(C // L):
                rows_v[b, r, pl.ds(j * L, L)] = (
                    rows_v[b, r, pl.ds(j * L, L)] * bex)

    def start_scatter(b):
        pltpu.async_copy(rows_v.at[b], acc_sh.at[dst_v.at[b]], ssem, add=True)

    def wait_scatter():
        pltpu.make_async_copy(rows_v.at[0], acc_sh.at[dst_v.at[0]], ssem).wait()

    def superblock(S, carry):
        sbase = ebase + S * SB * K
        pltpu.sync_copy(src_hbm.at[pl.ds(sbase, SB * K)], src_blk)
        pltpu.sync_copy(dst_hbm.at[pl.ds(sbase, SB * K)], dst_blk)
        start_gather(0, 0)
        start_gather(1, 1)

        def inner(I, carry2):
            for jj in range(NB):
                c = I * NB + jj
                wait_gather(c, jj)
                scale(c, jj)

                @pl.when(c + 2 < SB)
                def _pf():
                    start_gather(c + 2, (jj + 2) % NB)

                if jj == 0:
                    @pl.when(I > 0)
                    def _sc0():
                        wait_scatter()
                        start_scatter(NB - 1)
                elif jj == 1:
                    @pl.when((S > 0) | (I > 0))
                    def _w1():
                        wait_scatter()
                    start_scatter(0)
                else:
                    wait_scatter()
                    start_scatter(jj - 1)
            return carry2

        lax.fori_loop(0, SB // NB, inner, 0)
        wait_scatter()
        start_scatter(NB - 1)
        return carry

    lax.fori_loop(0, NSB, superblock, 0)
    wait_scatter()

    plsc.subcore_barrier()

    pltpu.sync_copy(acc_sh.at[pl.ds(row0, RPT)], acc_hbm.at[cid, pl.ds(row0, RPT)])

    @pl.when(sid == 0)
    def _write_tail():
        pltpu.sync_copy(acc_sh.at[pl.ds(TAIL, REM)],
                        acc_hbm.at[cid, pl.ds(TAIL, REM)])

    for p in range(NPH):
        pb = p * PH
        pltpu.sync_copy(denom_v.at[pl.ds(pb, PH)], den_sh.at[pl.ds(sid * PH, PH)])
        plsc.subcore_barrier()
        my0 = pb + sid * PR
        pltpu.sync_copy(den_sh.at[pl.ds(sid * PR, PR)], dacc_v)
        for k in range(1, NS):
            pltpu.sync_copy(den_sh.at[pl.ds(k * PH + sid * PR, PR)], dstage_v)

            def add_body(i, carry):
                dacc_v[pl.ds(i * L, L)] = (
                    dacc_v[pl.ds(i * L, L)] + dstage_v[pl.ds(i * L, L)])
                return carry

            lax.fori_loop(0, RV, add_body, 0)
        pltpu.sync_copy(dacc_v, den_hbm.at[pl.ds(cid * N + my0, PR)])
        plsc.subcore_barrier()

    pltpu.sync_copy(denom_v.at[pl.ds(TAIL, REM)], den_sh.at[pl.ds(sid * REM, REM)])
    plsc.subcore_barrier()

    @pl.when(sid == 0)
    def _den_tail():
        pltpu.sync_copy(den_sh.at[pl.ds(0, REM)], t0_v)
        for k in range(1, NS):
            pltpu.sync_copy(den_sh.at[pl.ds(k * REM, REM)], t1_v)
            t0_v[...] = t0_v[...] + t1_v[...]
        pltpu.sync_copy(t0_v, den_hbm.at[pl.ds(cid * N + TAIL, REM)])


_sc_call = pl.kernel(
    _sc_body,
    out_type=[
        jax.ShapeDtypeStruct((NC, N, C), jnp.float32),
        jax.ShapeDtypeStruct((NC * N,), jnp.float32),
    ],
    mesh=plsc.VectorSubcoreMesh(
        core_axis_name="c", subcore_axis_name="s", num_cores=NC, num_subcores=NS
    ),
    compiler_params=pltpu.CompilerParams(needs_layout_passes=False),
    scratch_types=[
        pltpu.VMEM((N,), jnp.float32),
        pltpu.VMEM((N,), jnp.float32),
        pltpu.VMEM((L,), jnp.float32),
        pltpu.VMEM((SB * K,), jnp.int32),
        pltpu.VMEM((SB * K,), jnp.int32),
        pltpu.VMEM((NB, K), jnp.int32),
        pltpu.VMEM((2 * L,), jnp.float32),
        pltpu.VMEM((NB, K, C), jnp.float32),
        pltpu.VMEM((N,), jnp.float32),
        pltpu.VMEM((PR,), jnp.float32),
        pltpu.VMEM((PR,), jnp.float32),
        pltpu.VMEM((L,), jnp.float32),
        pltpu.VMEM((L,), jnp.float32),
        pltpu.VMEM_SHARED((N, C), jnp.float32),
        pltpu.VMEM_SHARED((NS * PH,), jnp.float32),
        pltpu.SemaphoreType.DMA((NB,)),
        pltpu.SemaphoreType.DMA,
    ],
)


def _final_body(acc_ref, den_ref, bias_ref, out_ref):
    a = acc_ref[0] + acc_ref[1]
    den = den_ref[0] + den_ref[1]
    o = a / (den + 1e-16) + bias_ref[...][None, :]
    m = jnp.max(o, axis=1, keepdims=True)
    ls = o - m
    out_ref[...] = ls - jnp.log(jnp.sum(jnp.exp(ls), axis=1, keepdims=True))


_final_call = pl.pallas_call(
    _final_body,
    out_shape=jax.ShapeDtypeStruct((N, C), jnp.float32),
)


def kernel(x, W, att_src, att_dst, bias, edge_index):
    xp, as2, ad2, ms, md = _prep_call(x, W, att_src, att_dst)
    a_s = as2.reshape(N)
    a_d = ad2.reshape(N)
    mm = ms[0, 0] + md[0, 0]
    M = jnp.where(mm >= 0.0, mm, 0.2 * mm)
    m16 = jnp.full((L,), M, jnp.float32)
    src = edge_index[0]
    dst = edge_index[1]
    zrows = jnp.zeros((RPT, C), jnp.float32)
    zn = jnp.zeros((N,), jnp.float32)
    acc, den = _sc_call(xp, a_s, a_d, m16, src, dst, zrows, zn)
    den3 = den.reshape(NC, N, 1)
    return _final_call(acc, den3, bias)
